# baseline (speedup 1.0000x reference)
import functools

import jax
import jax.numpy as jnp
from jax import lax
from jax.experimental import pallas as pl
from jax.experimental.pallas import tpu as pltpu
from jax.experimental.pallas import tpu_sc as plsc

NC = 2
NS = 16
NW = NC * NS

B = 16384
D = 64
BW = B // NW
CH = 4
CW = BW // CH


def _k1_body(wi_hbm, wj_hbm, wt_hbm, wc_hbm, ei_hbm, ej_hbm,
             idxi_v, idxj_v, ei_v, ej_v, sem):
    wid = lax.axis_index("s") * NC + lax.axis_index("c")

    pltpu.sync_copy(wi_hbm.at[wid], idxi_v)
    pltpu.sync_copy(wj_hbm.at[wid], idxj_v)

    cps = [None, None]
    for j in range(CH):
        s = j & 1
        if cps[s] is not None:
            ca, cb, jo = cps[s]
            ca.wait()
            cb.wait()
            pltpu.sync_copy(ei_v.at[s], ei_hbm.at[pl.ds(wid * BW + jo * CW, CW)])
            pltpu.sync_copy(ej_v.at[s], ej_hbm.at[pl.ds(wid * BW + jo * CW, CW)])
        cps[s] = (
            pltpu.async_copy(wt_hbm.at[idxi_v.at[j]], ei_v.at[s], sem),
            pltpu.async_copy(wc_hbm.at[idxj_v.at[j]], ej_v.at[s], sem),
            j,
        )
    for s in range(2):
        ca, cb, jo = cps[s]
        ca.wait()
        cb.wait()
        pltpu.sync_copy(ei_v.at[s], ei_hbm.at[pl.ds(wid * BW + jo * CW, CW)])
        pltpu.sync_copy(ej_v.at[s], ej_hbm.at[pl.ds(wid * BW + jo * CW, CW)])


@functools.lru_cache(maxsize=1)
def _get_k1():
    return pl.kernel(
        _k1_body,
        out_type=[
            jax.ShapeDtypeStruct((B, 2 * D), jnp.float32),
            jax.ShapeDtypeStruct((B, 2 * D), jnp.float32),
        ],
        mesh=plsc.VectorSubcoreMesh(core_axis_name="c", subcore_axis_name="s"),
        compiler_params=pltpu.CompilerParams(
            needs_layout_passes=False, use_tc_tiling_on_sc=True,
            disable_bounds_checks=True),
        scratch_types=[
            pltpu.VMEM((CH, CW), jnp.int32),
            pltpu.VMEM((CH, CW), jnp.int32),
            pltpu.VMEM((2, CW, 2 * D), jnp.float32),
            pltpu.VMEM((2, CW, 2 * D), jnp.float32),
            pltpu.SemaphoreType.DMA,
        ],
    )


G = 8
BR = B // G


def _k2a_body(eif_ref, ejf_ref, pi_ref, pj_ref, yp_ref):
    ei = jnp.where(pi_ref[...] == 1, eif_ref[..., D:], eif_ref[..., :D])
    ej = jnp.where(pj_ref[...] == 1, ejf_ref[..., D:], ejf_ref[..., :D])
    yp_ref[...] = jnp.sum(ei * ej, axis=1, keepdims=True)


def _k2b_body(yp_ref, yt_ref, o_ref):
    yp = yp_ref[...]
    ws = jnp.sum(jnp.power(yp * jnp.float32(0.01), jnp.float32(0.75)))
    d = yp - jnp.log(yt_ref[...].astype(jnp.float32))
    o_ref[...] = ws * (d * d)


def kernel(w_i, w_j, y_true, W_target, W_context):
    v = W_target.shape[0]
    qi3 = (w_i >> 1).reshape(NW, CH, CW)
    qj3 = (w_j >> 1).reshape(NW, CH, CW)
    wt2 = W_target.reshape(v // 2, 2 * D)
    wc2 = W_context.reshape(v // 2, 2 * D)
    eif, ejf = _get_k1()(qi3, qj3, wt2, wc2)
    ypred = pl.pallas_call(
        _k2a_body,
        grid=(G,),
        in_specs=[
            pl.BlockSpec((BR, 2 * D), lambda i: (i, 0)),
            pl.BlockSpec((BR, 2 * D), lambda i: (i, 0)),
            pl.BlockSpec((BR, 1), lambda i: (i, 0)),
            pl.BlockSpec((BR, 1), lambda i: (i, 0)),
        ],
        out_specs=pl.BlockSpec((BR, 1), lambda i: (i, 0)),
        out_shape=jax.ShapeDtypeStruct((B, 1), jnp.float32),
    )(eif, ejf, w_i & 1, w_j & 1)
    return pl.pallas_call(
        _k2b_body,
        out_shape=jax.ShapeDtypeStruct((B, 1), jnp.float32),
    )(ypred, y_true)


# speedup vs baseline: 1.2317x; 1.2317x over previous
import functools

import jax
import jax.numpy as jnp
from jax import lax
from jax.experimental import pallas as pl
from jax.experimental.pallas import tpu as pltpu
from jax.experimental.pallas import tpu_sc as plsc

NC = 2
NS = 16
NW = NC * NS

B = 16384
D = 64
BW = B // NW
CH = 4
CW = BW // CH


def _k1_body(wi_hbm, wj_hbm, w2_hbm, ei_hbm, ej_hbm,
             idxi_v, idxj_v, ei_v, ej_v, sem):
    wid = lax.axis_index("s") * NC + lax.axis_index("c")

    pltpu.sync_copy(wi_hbm.at[wid], idxi_v)
    pltpu.sync_copy(wj_hbm.at[wid], idxj_v)

    cps = [None, None]
    for j in range(CH):
        s = j & 1
        if cps[s] is not None:
            ca, cb, jo = cps[s]
            ca.wait()
            cb.wait()
            pltpu.sync_copy(ei_v.at[s], ei_hbm.at[pl.ds(wid * BW + jo * CW, CW)])
            pltpu.sync_copy(ej_v.at[s], ej_hbm.at[pl.ds(wid * BW + jo * CW, CW)])
        cps[s] = (
            pltpu.async_copy(w2_hbm.at[idxi_v.at[j]], ei_v.at[s], sem),
            pltpu.async_copy(w2_hbm.at[idxj_v.at[j]], ej_v.at[s], sem),
            j,
        )
    for s in range(2):
        ca, cb, jo = cps[s]
        ca.wait()
        cb.wait()
        pltpu.sync_copy(ei_v.at[s], ei_hbm.at[pl.ds(wid * BW + jo * CW, CW)])
        pltpu.sync_copy(ej_v.at[s], ej_hbm.at[pl.ds(wid * BW + jo * CW, CW)])


@functools.lru_cache(maxsize=1)
def _get_k1():
    return pl.kernel(
        _k1_body,
        out_type=[
            jax.ShapeDtypeStruct((B, 2 * D), jnp.float32),
            jax.ShapeDtypeStruct((B, 2 * D), jnp.float32),
        ],
        mesh=plsc.VectorSubcoreMesh(core_axis_name="c", subcore_axis_name="s"),
        compiler_params=pltpu.CompilerParams(
            needs_layout_passes=False, use_tc_tiling_on_sc=True,
            disable_bounds_checks=True),
        scratch_types=[
            pltpu.VMEM((CH, CW), jnp.int32),
            pltpu.VMEM((CH, CW), jnp.int32),
            pltpu.VMEM((2, CW, 2 * D), jnp.float32),
            pltpu.VMEM((2, CW, 2 * D), jnp.float32),
            pltpu.SemaphoreType.DMA,
        ],
    )


G = 8
BR = B // G


def _k2a_body(eif_ref, ejf_ref, yp_ref):
    yp_ref[...] = jnp.sum(eif_ref[..., :D] * ejf_ref[..., D:], axis=1,
                          keepdims=True)


def _k2b_body(yp_ref, yt_ref, o_ref):
    yp = yp_ref[...]
    ws = jnp.sum(jnp.power(yp * jnp.float32(0.01), jnp.float32(0.75)))
    d = yp - jnp.log(yt_ref[...].astype(jnp.float32))
    o_ref[...] = ws * (d * d)


def kernel(w_i, w_j, y_true, W_target, W_context):
    wi3 = w_i.reshape(NW, CH, CW)
    wj3 = w_j.reshape(NW, CH, CW)
    w2 = jnp.concatenate([W_target, W_context], axis=1)
    eif, ejf = _get_k1()(wi3, wj3, w2)
    ypred = pl.pallas_call(
        _k2a_body,
        grid=(G,),
        in_specs=[
            pl.BlockSpec((BR, 2 * D), lambda i: (i, 0)),
            pl.BlockSpec((BR, 2 * D), lambda i: (i, 0)),
        ],
        out_specs=pl.BlockSpec((BR, 1), lambda i: (i, 0)),
        out_shape=jax.ShapeDtypeStruct((B, 1), jnp.float32),
    )(eif, ejf)
    return pl.pallas_call(
        _k2b_body,
        out_shape=jax.ShapeDtypeStruct((B, 1), jnp.float32),
    )(ypred, y_true)
